# Initial kernel scaffold; baseline (speedup 1.0000x reference)
import functools

import jax
import jax.numpy as jnp
from jax import lax
from jax.experimental import pallas as pl
from jax.experimental.pallas import tpu as pltpu
from jax.experimental.pallas import tpu_sc as plsc

N = 10000
E = 320000
D = 128
L = 16
NC = 2
NS = 16
NW = NC * NS
EPW = E // NW
CH = 80
NCHUNK = EPW // CH
RPS = N // NS
ZC = 125


def _sc_partials(hidden, src2, dst2, w2):
    mesh = plsc.VectorSubcoreMesh(core_axis_name="c", subcore_axis_name="s")

    @functools.partial(
        pl.kernel,
        out_type=jax.ShapeDtypeStruct((NC, N, D), jnp.float32),
        mesh=mesh,
        scratch_types=[
            pltpu.VMEM_SHARED((N, D), jnp.float32),
            pltpu.VMEM((NCHUNK, CH), jnp.int32),
            pltpu.VMEM((NCHUNK, CH), jnp.int32),
            pltpu.VMEM((NCHUNK, CH), jnp.float32),
            pltpu.VMEM((CH, D), jnp.float32),
            pltpu.VMEM((ZC, D), jnp.float32),
            pltpu.SemaphoreType.DMA,
        ],
    )
    def sc_kernel(hidden_hbm, src_hbm, dst_hbm, w_hbm, out_hbm,
                  acc, src_v, dst_v, w_v, rows_v, zbuf, sem):
        cid = lax.axis_index("c")
        sid = lax.axis_index("s")
        wid = sid * NC + cid
        row_chunk_base = wid * NCHUNK

        zero = jnp.zeros((L,), jnp.float32)

        @pl.loop(0, ZC)
        def _(r):
            for j in range(D // L):
                zbuf[r, pl.ds(j * L, L)] = zero

        @pl.loop(0, RPS // ZC)
        def _(b):
            pltpu.sync_copy(zbuf, acc.at[pl.ds(sid * RPS + b * ZC, ZC)])

        pltpu.sync_copy(src_hbm.at[pl.ds(row_chunk_base, NCHUNK)], src_v)
        pltpu.sync_copy(dst_hbm.at[pl.ds(row_chunk_base, NCHUNK)], dst_v)
        pltpu.sync_copy(w_hbm.at[pl.ds(row_chunk_base, NCHUNK)], w_v)

        plsc.subcore_barrier()

        @pl.loop(0, NCHUNK)
        def _(k):
            pltpu.async_copy(hidden_hbm.at[src_v.at[k]], rows_v, sem).wait()

            @pl.loop(0, CH)
            def _(i):
                wvec = jnp.full((L,), w_v[k, i])
                for j in range(D // L):
                    sl = (i, pl.ds(j * L, L))
                    rows_v[sl] = rows_v[sl] * wvec

            pltpu.async_copy(rows_v, acc.at[dst_v.at[k]], sem, add=True).wait()

        plsc.subcore_barrier()

        @pl.loop(0, RPS // ZC)
        def _(b):
            r0 = sid * RPS + b * ZC
            pltpu.sync_copy(acc.at[pl.ds(r0, ZC)], zbuf)
            pltpu.sync_copy(zbuf, out_hbm.at[cid, pl.ds(r0, ZC)])

    return sc_kernel(hidden, src2, dst2, w2)


def _combine(hidden, p0, p1):
    BR = 1000

    def body(h_ref, a_ref, b_ref, o_ref):
        o_ref[...] = (h_ref[...] + a_ref[...] + b_ref[...]) * 0.5

    spec = pl.BlockSpec((BR, D), lambda i: (i, 0))
    return pl.pallas_call(
        body,
        grid=(N // BR,),
        in_specs=[spec, spec, spec],
        out_specs=spec,
        out_shape=jax.ShapeDtypeStruct((N, D), jnp.float32),
    )(hidden, p0, p1)


def kernel(hidden, edge_index, edge_weight):
    src2 = edge_index[1].reshape(E // CH, CH)
    dst2 = edge_index[0].reshape(E // CH, CH)
    w2 = edge_weight.reshape(E // CH, CH)
    p = _sc_partials(hidden, src2, dst2, w2)
    return _combine(hidden, p[0], p[1])


# speedup vs baseline: 2.8386x; 2.8386x over previous
import dataclasses
import functools

import jax
import jax.numpy as jnp
from jax import lax
from jax.experimental import pallas as pl
from jax.experimental.pallas import tpu as pltpu
from jax.experimental.pallas import tpu_sc as plsc

N = 10000
E = 320000
D = 128
L = 16
NC = 2
NS = 16
NW = NC * NS
CH = 128
NCHUNK = 80
EPW = NCHUNK * CH
EPAD = NW * EPW
RB = 80
NBLK = N // RB


def _sc_partials(hidden, src1, dst2, w1):
    mesh = plsc.VectorSubcoreMesh(core_axis_name="c", subcore_axis_name="s")

    cp = pltpu.CompilerParams()
    if "needs_layout_passes" in pltpu.CompilerParams.__dataclass_fields__:
        cp = dataclasses.replace(cp, needs_layout_passes=False)

    @functools.partial(
        pl.kernel,
        compiler_params=cp,
        out_type=jax.ShapeDtypeStruct((NC, N, D), jnp.float32),
        mesh=mesh,
        scratch_types=[
            pltpu.VMEM_SHARED((N, D), jnp.float32),
            pltpu.VMEM((EPW,), jnp.int32),
            pltpu.VMEM((NCHUNK, CH), jnp.int32),
            pltpu.VMEM((EPW,), jnp.float32),
            pltpu.VMEM((CH, D), jnp.float32),
            pltpu.SemaphoreType.DMA,
        ],
    )
    def sc_kernel(hidden_hbm, src_hbm, dst_hbm, w_hbm, out_hbm,
                  acc, src_v, dst_v, w_v, rows_v, sem):
        cid = lax.axis_index("c")
        sid = lax.axis_index("s")
        wid = sid * NC + cid

        zero = jnp.zeros((L,), jnp.float32)

        @pl.loop(0, CH)
        def _(r):
            for j in range(D // L):
                rows_v[r, pl.ds(j * L, L)] = zero

        @pl.loop(sid, NBLK, step=NS)
        def _(b):
            pltpu.sync_copy(rows_v.at[pl.ds(0, RB)], acc.at[pl.ds(b * RB, RB)])

        pltpu.sync_copy(src_hbm.at[pl.ds(wid * EPW, EPW)], src_v)
        pltpu.sync_copy(dst_hbm.at[pl.ds(wid * NCHUNK, NCHUNK)], dst_v)
        pltpu.sync_copy(w_hbm.at[pl.ds(wid * EPW, EPW)], w_v)

        plsc.subcore_barrier()

        @pl.loop(0, NCHUNK)
        def _(k):
            pltpu.async_copy(
                hidden_hbm.at[src_v.at[pl.ds(k * CH, CH)]], rows_v, sem
            ).wait()

            @pl.loop(0, CH)
            def _(i):
                wvec = plsc.load_gather(
                    w_v, [jnp.full((L,), k * CH + i, jnp.int32)])
                for j in range(D // L):
                    sl = (i, pl.ds(j * L, L))
                    rows_v[sl] = rows_v[sl] * wvec

            pltpu.async_copy(rows_v, acc.at[dst_v.at[k]], sem, add=True).wait()

        plsc.subcore_barrier()

        @pl.loop(sid, NBLK, step=NS)
        def _(b):
            pltpu.sync_copy(acc.at[pl.ds(b * RB, RB)], rows_v.at[pl.ds(0, RB)])
            pltpu.sync_copy(rows_v.at[pl.ds(0, RB)],
                            out_hbm.at[cid, pl.ds(b * RB, RB)])

    return sc_kernel(hidden, src1, dst2, w1)


def _combine(hidden, p0, p1):
    BR = 1000

    def body(h_ref, a_ref, b_ref, o_ref):
        o_ref[...] = (h_ref[...] + a_ref[...] + b_ref[...]) * 0.5

    spec = pl.BlockSpec((BR, D), lambda i: (i, 0))
    return pl.pallas_call(
        body,
        grid=(N // BR,),
        in_specs=[spec, spec, spec],
        out_specs=spec,
        out_shape=jax.ShapeDtypeStruct((N, D), jnp.float32),
    )(hidden, p0, p1)


def kernel(hidden, edge_index, edge_weight):
    pad = EPAD - E
    src1 = jnp.concatenate([edge_index[1], jnp.zeros((pad,), jnp.int32)])
    dst1 = jnp.concatenate([edge_index[0], jnp.zeros((pad,), jnp.int32)])
    w1 = jnp.concatenate([edge_weight, jnp.zeros((pad,), jnp.float32)])
    dst2 = dst1.reshape(NW * NCHUNK, CH)
    p = _sc_partials(hidden, src1, dst2, w1)
    return _combine(hidden, p[0], p[1])
